# baseline (speedup 1.0000x reference)
import jax
import jax.numpy as jnp
from jax.experimental import pallas as pl
from jax.experimental.pallas import tpu as pltpu

_K = 9
_EPS = 1e-5


def _fuse_kernel(ws_ref, bs_ref, wn_ref, bn_ref, wc_ref, bc_ref,
                 fs_ref, fn_ref, c0_ref):
    c = ws_ref.shape[0]
    wc1 = wc_ref[:, :c]
    wc2 = wc_ref[:, c:]
    fs_ref[...] = jnp.dot(wc1, ws_ref[...], preferred_element_type=jnp.float32)
    fn_ref[...] = jnp.dot(wc2, wn_ref[...], preferred_element_type=jnp.float32)
    c0_ref[...] = (bc_ref[...]
                   + jnp.dot(wc1, bs_ref[...], preferred_element_type=jnp.float32)
                   + jnp.dot(wc2, bn_ref[...], preferred_element_type=jnp.float32))


def _gram(xv):
    return jax.lax.dot_general(xv * -2.0, xv, (((0,), (0,)), ((), ())),
                               preferred_element_type=jnp.float32)


def _main_kernel(xb_ref, xn_ref, fs_ref, fn_ref, c0_ref, gamma_ref, beta_ref,
                 out_ref, g_scr, sq_scr, pre_scr, stats_scr, ss_scr):
    p = pl.program_id(0)
    b = pl.program_id(1)
    n_b = pl.num_programs(1)
    c, n = xb_ref.shape[1], xb_ref.shape[2]
    inf = jnp.float32(jnp.inf)

    @pl.when(p == 0)
    def _phase0():
        cur = jax.lax.rem(b, 2)
        nxt = jax.lax.rem(b + 1, 2)

        ones_c = jnp.ones((c, 1), jnp.float32)

        @pl.when(b == 0)
        def _prologue():
            xv = xb_ref[0]
            g_scr[0] = _gram(xv)
            sq_scr[0] = jax.lax.dot_general(
                xv * xv, ones_c, (((0,), (0,)), ((), ())),
                preferred_element_type=jnp.float32)

        xv_n = xn_ref[0]
        g_scr[nxt] = _gram(xv_n)
        sq_scr[nxt] = jax.lax.dot_general(
            xv_n * xv_n, ones_c, (((0,), (0,)), ((), ())),
            preferred_element_type=jnp.float32)

        e = sq_scr[cur] + g_scr[cur]
        mv = jnp.full((1, n), -inf, jnp.float32)
        for _ in range(_K):
            mv = jnp.min(jnp.where(e <= mv, inf, e), axis=0, keepdims=True)
        mt = (e <= mv).astype(jnp.bfloat16)

        xb16 = xb_ref[0].astype(jnp.bfloat16)
        nbt = jnp.dot(xb16, mt, preferred_element_type=jnp.float32)
        out_t = (jnp.dot(fs_ref[...].astype(jnp.bfloat16), xb16,
                         preferred_element_type=jnp.float32)
                 + jnp.dot(fn_ref[...].astype(jnp.bfloat16),
                           (nbt * (1.0 / _K)).astype(jnp.bfloat16),
                           preferred_element_type=jnp.float32)
                 + c0_ref[...])
        pre_scr[b] = out_t
        part = jnp.concatenate(
            [jnp.sum(out_t, axis=1, keepdims=True),
             jnp.sum(out_t * out_t, axis=1, keepdims=True)], axis=1)
        stats_scr[...] = jnp.where(b == 0, part, stats_scr[...] + part)

    @pl.when((p == 1) & (b == 0))
    def _finalize_stats():
        cnt = jnp.float32(n_b * n)
        mean = stats_scr[:, 0:1] / cnt
        var = stats_scr[:, 1:2] / cnt - mean * mean
        inv = jax.lax.rsqrt(var + _EPS)
        scale = gamma_ref[...] * inv
        ss_scr[...] = jnp.concatenate(
            [scale, beta_ref[...] - mean * scale], axis=1)

    @pl.when(p == 1)
    def _phase1():
        scale = ss_scr[:, 0:1]
        shift = ss_scr[:, 1:2]
        out_ref[0] = jnp.maximum(pre_scr[b] * scale + shift + xb_ref[0], 0.0)


def kernel(x, W_self, b_self, W_nb, b_nb, W_comb, b_comb, gamma, beta):
    B, C, H, W = x.shape
    N = H * W
    xr = x.reshape(B, C, N)
    f_self, f_nb, c0 = pl.pallas_call(
        _fuse_kernel,
        out_shape=(
            jax.ShapeDtypeStruct((C, C), jnp.float32),
            jax.ShapeDtypeStruct((C, C), jnp.float32),
            jax.ShapeDtypeStruct((C, 1), jnp.float32),
        ),
    )(W_self, b_self[:, None], W_nb, b_nb[:, None], W_comb, b_comb[:, None])

    out = pl.pallas_call(
        _main_kernel,
        grid=(2, B),
        in_specs=[
            pl.BlockSpec((1, C, N), lambda p, b: (b, 0, 0)),
            pl.BlockSpec((1, C, N),
                         lambda p, b: (jnp.where(
                             p == 0, jnp.minimum(b + 1, B - 1), B - 1), 0, 0)),
            pl.BlockSpec((C, C), lambda p, b: (0, 0)),
            pl.BlockSpec((C, C), lambda p, b: (0, 0)),
            pl.BlockSpec((C, 1), lambda p, b: (0, 0)),
            pl.BlockSpec((C, 1), lambda p, b: (0, 0)),
            pl.BlockSpec((C, 1), lambda p, b: (0, 0)),
        ],
        out_specs=pl.BlockSpec((1, C, N),
                               lambda p, b: (jnp.where(p == 0, 0, b), 0, 0)),
        out_shape=jax.ShapeDtypeStruct((B, C, N), jnp.float32),
        scratch_shapes=[
            pltpu.VMEM((2, N, N), jnp.float32),
            pltpu.VMEM((2, N, 1), jnp.float32),
            pltpu.VMEM((B, C, N), jnp.float32),
            pltpu.VMEM((C, 2), jnp.float32),
            pltpu.VMEM((C, 2), jnp.float32),
        ],
    )(xr, xr, f_self, f_nb, c0, gamma[:, None], beta[:, None])
    return out.reshape(B, C, H, W)


# speedup vs baseline: 1.0478x; 1.0478x over previous
import jax
import jax.numpy as jnp
from jax.experimental import pallas as pl
from jax.experimental.pallas import tpu as pltpu

_K = 9
_EPS = 1e-5


def _fuse_kernel(ws_ref, bs_ref, wn_ref, bn_ref, wc_ref, bc_ref,
                 fs_ref, fn_ref, c0_ref):
    c = ws_ref.shape[0]
    wc1 = wc_ref[:, :c]
    wc2 = wc_ref[:, c:]
    fs_ref[...] = jnp.dot(wc1, ws_ref[...], preferred_element_type=jnp.float32)
    fn_ref[...] = jnp.dot(wc2, wn_ref[...], preferred_element_type=jnp.float32)
    c0_ref[...] = (bc_ref[...]
                   + jnp.dot(wc1, bs_ref[...], preferred_element_type=jnp.float32)
                   + jnp.dot(wc2, bn_ref[...], preferred_element_type=jnp.float32))


def _gram(xv):
    return jax.lax.dot_general(xv * -2.0, xv, (((0,), (0,)), ((), ())),
                               preferred_element_type=jnp.float32)


def _main_kernel(xb_ref, xn_ref, fs_ref, fn_ref, c0_ref, gamma_ref, beta_ref,
                 out_ref, g_scr, sq_scr, pre_scr, stats_scr, ss_scr):
    p = pl.program_id(0)
    b = pl.program_id(1)
    n_b = pl.num_programs(1)
    c, n = xb_ref.shape[1], xb_ref.shape[2]
    inf = jnp.float32(jnp.inf)

    @pl.when(p == 0)
    def _phase0():
        cur = jax.lax.rem(b, 2)
        nxt = jax.lax.rem(b + 1, 2)

        @pl.when(b == 0)
        def _prologue():
            xv = xb_ref[0]
            g_scr[0] = _gram(xv)
            sq_scr[0] = jnp.sum(xv * xv, axis=0, keepdims=True)

        xv_n = xn_ref[0]
        g_scr[nxt] = _gram(xv_n)
        sq_scr[nxt] = jnp.sum(xv_n * xv_n, axis=0, keepdims=True)

        e = sq_scr[cur] + g_scr[cur]
        mv = jnp.full((n, 1), -inf, jnp.float32)
        for _ in range(_K):
            mv = jnp.min(jnp.where(e <= mv, inf, e), axis=1, keepdims=True)
        m = (e <= mv).astype(jnp.bfloat16)

        xb16 = xb_ref[0].astype(jnp.bfloat16)
        nbt = jax.lax.dot_general(xb16, m, (((1,), (1,)), ((), ())),
                                  preferred_element_type=jnp.float32)
        out_t = (jnp.dot(fs_ref[...].astype(jnp.bfloat16), xb16,
                         preferred_element_type=jnp.float32)
                 + jnp.dot(fn_ref[...].astype(jnp.bfloat16),
                           (nbt * (1.0 / _K)).astype(jnp.bfloat16),
                           preferred_element_type=jnp.float32)
                 + c0_ref[...])
        pre_scr[b] = out_t
        part = jnp.concatenate(
            [jnp.sum(out_t, axis=1, keepdims=True),
             jnp.sum(out_t * out_t, axis=1, keepdims=True)], axis=1)
        stats_scr[...] = jnp.where(b == 0, part, stats_scr[...] + part)

    @pl.when((p == 1) & (b == 0))
    def _finalize_stats():
        cnt = jnp.float32(n_b * n)
        mean = stats_scr[:, 0:1] / cnt
        var = stats_scr[:, 1:2] / cnt - mean * mean
        inv = jax.lax.rsqrt(var + _EPS)
        scale = gamma_ref[...] * inv
        ss_scr[...] = jnp.concatenate(
            [scale, beta_ref[...] - mean * scale], axis=1)

    @pl.when(p == 1)
    def _phase1():
        scale = ss_scr[:, 0:1]
        shift = ss_scr[:, 1:2]
        out_ref[0] = jnp.maximum(pre_scr[b] * scale + shift + xb_ref[0], 0.0)


def kernel(x, W_self, b_self, W_nb, b_nb, W_comb, b_comb, gamma, beta):
    B, C, H, W = x.shape
    N = H * W
    xr = x.reshape(B, C, N)
    f_self, f_nb, c0 = pl.pallas_call(
        _fuse_kernel,
        out_shape=(
            jax.ShapeDtypeStruct((C, C), jnp.float32),
            jax.ShapeDtypeStruct((C, C), jnp.float32),
            jax.ShapeDtypeStruct((C, 1), jnp.float32),
        ),
    )(W_self, b_self[:, None], W_nb, b_nb[:, None], W_comb, b_comb[:, None])

    out = pl.pallas_call(
        _main_kernel,
        grid=(2, B),
        in_specs=[
            pl.BlockSpec((1, C, N), lambda p, b: (b, 0, 0)),
            pl.BlockSpec((1, C, N),
                         lambda p, b: (jnp.where(
                             p == 0, jnp.minimum(b + 1, B - 1), B - 1), 0, 0)),
            pl.BlockSpec((C, C), lambda p, b: (0, 0)),
            pl.BlockSpec((C, C), lambda p, b: (0, 0)),
            pl.BlockSpec((C, 1), lambda p, b: (0, 0)),
            pl.BlockSpec((C, 1), lambda p, b: (0, 0)),
            pl.BlockSpec((C, 1), lambda p, b: (0, 0)),
        ],
        out_specs=pl.BlockSpec((1, C, N),
                               lambda p, b: (jnp.where(p == 0, 0, b), 0, 0)),
        out_shape=jax.ShapeDtypeStruct((B, C, N), jnp.float32),
        scratch_shapes=[
            pltpu.VMEM((2, N, N), jnp.float32),
            pltpu.VMEM((2, 1, N), jnp.float32),
            pltpu.VMEM((B, C, N), jnp.float32),
            pltpu.VMEM((C, 2), jnp.float32),
            pltpu.VMEM((C, 2), jnp.float32),
        ],
    )(xr, xr, f_self, f_nb, c0, gamma[:, None], beta[:, None])
    return out.reshape(B, C, H, W)


# speedup vs baseline: 1.1010x; 1.0507x over previous
import jax
import jax.numpy as jnp
from jax.experimental import pallas as pl
from jax.experimental.pallas import tpu as pltpu

_K = 9
_EPS = 1e-5


def _fuse_kernel(ws_ref, bs_ref, wn_ref, bn_ref, wc_ref, bc_ref,
                 fs_ref, fn_ref, c0_ref):
    c = ws_ref.shape[0]
    wc1 = wc_ref[:, :c]
    wc2 = wc_ref[:, c:]
    fs_ref[...] = jnp.dot(wc1, ws_ref[...], preferred_element_type=jnp.float32)
    fn_ref[...] = jnp.dot(wc2, wn_ref[...], preferred_element_type=jnp.float32)
    c0_ref[...] = (bc_ref[...]
                   + jnp.dot(wc1, bs_ref[...], preferred_element_type=jnp.float32)
                   + jnp.dot(wc2, bn_ref[...], preferred_element_type=jnp.float32))


def _main_kernel(xf_ref, fs_ref, fn_ref, c0_ref, gamma_ref, beta_ref,
                 out_ref, pre_scr, stats_scr):
    b = pl.program_id(0)
    n_b = pl.num_programs(0) - 1
    n = xf_ref.shape[2]
    inf = jnp.float32(jnp.inf)

    @pl.when(b < n_b)
    def _compute():
        xv = xf_ref[b]
        g = jax.lax.dot_general(xv * -2.0, xv, (((0,), (0,)), ((), ())),
                                preferred_element_type=jnp.float32)
        sq = jnp.sum(xv * xv, axis=0, keepdims=True)
        e = sq + g
        mv = jnp.full((n, 1), -inf, jnp.float32)
        for _ in range(_K):
            mv = jnp.min(jnp.where(e <= mv, inf, e), axis=1, keepdims=True)
        m = (e <= mv).astype(jnp.bfloat16)

        xb16 = xv.astype(jnp.bfloat16)
        nbt = jax.lax.dot_general(xb16, m, (((1,), (1,)), ((), ())),
                                  preferred_element_type=jnp.float32)
        out_t = (jnp.dot(fs_ref[...].astype(jnp.bfloat16), xb16,
                         preferred_element_type=jnp.float32)
                 + jnp.dot(fn_ref[...].astype(jnp.bfloat16),
                           (nbt * (1.0 / _K)).astype(jnp.bfloat16),
                           preferred_element_type=jnp.float32)
                 + c0_ref[...])
        pre_scr[b] = out_t
        part = jnp.concatenate(
            [jnp.sum(out_t, axis=1, keepdims=True),
             jnp.sum(out_t * out_t, axis=1, keepdims=True)], axis=1)
        stats_scr[...] = jnp.where(b == 0, part, stats_scr[...] + part)

    @pl.when(b == n_b)
    def _finalize():
        cnt = jnp.float32(n_b * n)
        mean = stats_scr[:, 0:1] / cnt
        var = stats_scr[:, 1:2] / cnt - mean * mean
        inv = jax.lax.rsqrt(var + _EPS)
        scale = (gamma_ref[...] * inv)[None]
        shift = (beta_ref[...] - mean * gamma_ref[...] * inv)[None]
        out_ref[...] = jnp.maximum(
            pre_scr[...] * scale + shift + xf_ref[...], 0.0)


def kernel(x, W_self, b_self, W_nb, b_nb, W_comb, b_comb, gamma, beta):
    B, C, H, W = x.shape
    N = H * W
    xr = x.reshape(B, C, N)
    f_self, f_nb, c0 = pl.pallas_call(
        _fuse_kernel,
        out_shape=(
            jax.ShapeDtypeStruct((C, C), jnp.float32),
            jax.ShapeDtypeStruct((C, C), jnp.float32),
            jax.ShapeDtypeStruct((C, 1), jnp.float32),
        ),
    )(W_self, b_self[:, None], W_nb, b_nb[:, None], W_comb, b_comb[:, None])

    out = pl.pallas_call(
        _main_kernel,
        grid=(B + 1,),
        in_specs=[
            pl.BlockSpec((B, C, N), lambda b: (0, 0, 0)),
            pl.BlockSpec((C, C), lambda b: (0, 0)),
            pl.BlockSpec((C, C), lambda b: (0, 0)),
            pl.BlockSpec((C, 1), lambda b: (0, 0)),
            pl.BlockSpec((C, 1), lambda b: (0, 0)),
            pl.BlockSpec((C, 1), lambda b: (0, 0)),
        ],
        out_specs=pl.BlockSpec((B, C, N), lambda b: (0, 0, 0)),
        out_shape=jax.ShapeDtypeStruct((B, C, N), jnp.float32),
        scratch_shapes=[
            pltpu.VMEM((B, C, N), jnp.float32),
            pltpu.VMEM((C, 2), jnp.float32),
        ],
    )(xr, f_self, f_nb, c0, gamma[:, None], beta[:, None])
    return out.reshape(B, C, H, W)
